# Initial kernel scaffold; baseline (speedup 1.0000x reference)
import functools

import jax
import jax.numpy as jnp
from jax import lax
from jax.experimental import pallas as pl
from jax.experimental.pallas import tpu as pltpu
from jax.experimental.pallas import tpu_sc as plsc

NUM_SC = 2
NUM_TILES = 16
NUM_WORKERS = NUM_SC * NUM_TILES
DEG_W = 16


def _sc_mesh():
    return plsc.VectorSubcoreMesh(core_axis_name="c", subcore_axis_name="s")


def _sc_degree(dst3, n_nodes):
    _, nchunk, c = dst3.shape
    rpt = n_nodes // NUM_TILES

    @functools.partial(
        pl.kernel,
        out_type=jax.ShapeDtypeStruct((NUM_SC, n_nodes, DEG_W), jnp.float32),
        mesh=_sc_mesh(),
        scratch_types=[
            pltpu.VMEM((nchunk, c), jnp.int32),
            pltpu.VMEM((c, DEG_W), jnp.float32),
            pltpu.VMEM_SHARED((n_nodes, DEG_W), jnp.float32),
        ],
    )
    def k(dst_hbm, ones_hbm, zeros_hbm, out_hbm, dsti_v, ones_v, acc_sh):
        cid = lax.axis_index("c")
        sid = lax.axis_index("s")
        wid = sid * NUM_SC + cid
        pltpu.sync_copy(dst_hbm.at[wid], dsti_v)
        pltpu.sync_copy(ones_hbm, ones_v)
        pltpu.sync_copy(zeros_hbm, acc_sh.at[pl.ds(sid * rpt, rpt)])
        plsc.subcore_barrier()

        def body(j, carry):
            pltpu.sync_copy(ones_v, acc_sh.at[dsti_v.at[j]], add=True)
            return carry

        lax.fori_loop(0, nchunk, body, 0)
        plsc.subcore_barrier()
        pltpu.sync_copy(
            acc_sh.at[pl.ds(sid * rpt, rpt)],
            out_hbm.at[cid, pl.ds(sid * rpt, rpt)],
        )

    ones = jnp.ones((c, DEG_W), jnp.float32)
    zeros = jnp.zeros((rpt, DEG_W), jnp.float32)
    return k(dst3, ones, zeros)


def _sc_aggregate(h, src3, dst3):
    n_nodes, d = h.shape
    _, nchunk, c = src3.shape
    rpt = n_nodes // NUM_TILES

    @functools.partial(
        pl.kernel,
        out_type=jax.ShapeDtypeStruct((NUM_SC, n_nodes, d), jnp.float32),
        mesh=_sc_mesh(),
        scratch_types=[
            pltpu.VMEM((nchunk, c), jnp.int32),
            pltpu.VMEM((nchunk, c), jnp.int32),
            pltpu.VMEM((2, c, d), jnp.float32),
            pltpu.VMEM_SHARED((n_nodes, d), jnp.float32),
            pltpu.SemaphoreType.DMA,
            pltpu.SemaphoreType.DMA,
        ],
    )
    def k(h_hbm, src_hbm, dst_hbm, zeros_hbm, out_hbm,
          srci_v, dsti_v, rows_v, acc_sh, sem0, sem1):
        cid = lax.axis_index("c")
        sid = lax.axis_index("s")
        wid = sid * NUM_SC + cid
        pltpu.sync_copy(src_hbm.at[wid], srci_v)
        pltpu.sync_copy(dst_hbm.at[wid], dsti_v)
        pltpu.sync_copy(zeros_hbm, acc_sh.at[pl.ds(sid * rpt, rpt)])
        pltpu.async_copy(h_hbm.at[srci_v.at[0]], rows_v.at[0], sem0)
        pltpu.async_copy(h_hbm.at[srci_v.at[1]], rows_v.at[1], sem1)
        plsc.subcore_barrier()

        sems = (sem0, sem1)

        def body(jh, carry):
            for b in range(2):
                jj = 2 * jh + b
                pltpu.make_async_copy(
                    h_hbm.at[srci_v.at[jj]], rows_v.at[b], sems[b]
                ).wait()
                pltpu.sync_copy(rows_v.at[b], acc_sh.at[dsti_v.at[jj]],
                                add=True)

                @pl.when(jj + 2 < nchunk)
                def _():
                    pltpu.async_copy(
                        h_hbm.at[srci_v.at[jj + 2]], rows_v.at[b], sems[b]
                    )
            return carry

        lax.fori_loop(0, nchunk // 2, body, 0)
        plsc.subcore_barrier()
        pltpu.sync_copy(
            acc_sh.at[pl.ds(sid * rpt, rpt)],
            out_hbm.at[cid, pl.ds(sid * rpt, rpt)],
        )

    zeros = jnp.zeros((rpt, d), jnp.float32)
    return k(h, src3, dst3, zeros)


def _tc_layer1(x, w1, d0, d1, blk):
    n, d_in = x.shape
    d_h = w1.shape[1]

    def body(x_ref, w_ref, d0_ref, d1_ref, hp_ref, dis_ref):
        deg = d0_ref[...] + d1_ref[...] + 1.0
        dis = lax.rsqrt(deg)
        h = jnp.dot(x_ref[...], w_ref[...],
                    preferred_element_type=jnp.float32,
                    precision=lax.Precision.HIGHEST)
        hp_ref[...] = h * dis
        dis_ref[...] = dis

    return pl.pallas_call(
        body,
        grid=(n // blk,),
        in_specs=[
            pl.BlockSpec((blk, d_in), lambda i: (i, 0)),
            pl.BlockSpec((d_in, d_h), lambda i: (0, 0)),
            pl.BlockSpec((blk, 1), lambda i: (i, 0)),
            pl.BlockSpec((blk, 1), lambda i: (i, 0)),
        ],
        out_specs=[
            pl.BlockSpec((blk, d_h), lambda i: (i, 0)),
            pl.BlockSpec((blk, 1), lambda i: (i, 0)),
        ],
        out_shape=[
            jax.ShapeDtypeStruct((n, d_h), jnp.float32),
            jax.ShapeDtypeStruct((n, 1), jnp.float32),
        ],
    )(x, w1, d0, d1)


def _tc_layer2(p, hp, dis, b1, w2, blk):
    n, d_h = hp.shape
    d_o = w2.shape[1]

    def body(p_ref, hp_ref, dis_ref, b1_ref, w2_ref, gp_ref):
        s = p_ref[0] + p_ref[1] + hp_ref[...]
        z = jnp.maximum(dis_ref[...] * s + b1_ref[...], 0.0)
        g = jnp.dot(z, w2_ref[...],
                    preferred_element_type=jnp.float32,
                    precision=lax.Precision.HIGHEST)
        gp_ref[...] = g * dis_ref[...]

    return pl.pallas_call(
        body,
        grid=(n // blk,),
        in_specs=[
            pl.BlockSpec((NUM_SC, blk, d_h), lambda i: (0, i, 0)),
            pl.BlockSpec((blk, d_h), lambda i: (i, 0)),
            pl.BlockSpec((blk, 1), lambda i: (i, 0)),
            pl.BlockSpec((1, d_h), lambda i: (0, 0)),
            pl.BlockSpec((d_h, d_o), lambda i: (0, 0)),
        ],
        out_specs=pl.BlockSpec((blk, d_o), lambda i: (i, 0)),
        out_shape=jax.ShapeDtypeStruct((n, d_o), jnp.float32),
    )(p, hp, dis, b1, w2)


def _tc_final(q, gp, dis, b2, blk):
    n, d_o = gp.shape

    def body(q_ref, gp_ref, dis_ref, b2_ref, out_ref):
        s = q_ref[0] + q_ref[1] + gp_ref[...]
        out_ref[...] = dis_ref[...] * s + b2_ref[...]

    return pl.pallas_call(
        body,
        grid=(n // blk,),
        in_specs=[
            pl.BlockSpec((NUM_SC, blk, d_o), lambda i: (0, i, 0)),
            pl.BlockSpec((blk, d_o), lambda i: (i, 0)),
            pl.BlockSpec((blk, 1), lambda i: (i, 0)),
            pl.BlockSpec((1, d_o), lambda i: (0, 0)),
        ],
        out_specs=pl.BlockSpec((blk, d_o), lambda i: (i, 0)),
        out_shape=jax.ShapeDtypeStruct((n, d_o), jnp.float32),
    )(q, gp, dis, b2)


def kernel(x, edge_index, W1, b1, W2, b2):
    n, _ = x.shape
    e = edge_index.shape[1]

    epw = e // NUM_WORKERS
    c = 100 if epw % 100 == 0 else 80
    nchunk = epw // c
    assert epw * NUM_WORKERS == e and nchunk * c == epw and nchunk % 2 == 0

    ei = edge_index.astype(jnp.int32)
    src3 = ei[0].reshape(NUM_WORKERS, nchunk, c)
    dst3 = ei[1].reshape(NUM_WORKERS, nchunk, c)

    degp = _sc_degree(dst3, n)
    d0 = degp[0, :, 0:1]
    d1 = degp[1, :, 0:1]

    blk = 1000 if n % 1000 == 0 else 8
    hp, dis = _tc_layer1(x, W1, d0, d1, blk)
    p = _sc_aggregate(hp, src3, dst3)
    gp = _tc_layer2(p, hp, dis, b1.reshape(1, -1), W2, blk)
    q = _sc_aggregate(gp, src3, dst3)
    return _tc_final(q, gp, dis, b2.reshape(1, -1), blk)


# speedup vs baseline: 26.7230x; 26.7230x over previous
import functools

import jax
import jax.numpy as jnp
from jax import lax
from jax.experimental import pallas as pl
from jax.experimental.pallas import tpu as pltpu
from jax.experimental.pallas import tpu_sc as plsc

NUM_SC = 2
NUM_TILES = 16
NUM_WORKERS = NUM_SC * NUM_TILES
DEG_W = 8


def _sc_mesh():
    return plsc.VectorSubcoreMesh(core_axis_name="c", subcore_axis_name="s")


_SC_PARAMS = pltpu.CompilerParams(use_tc_tiling_on_sc=False)


def _sc_degree(dst3, n_pad):
    _, nchunk, c = dst3.shape
    rpt = n_pad // NUM_TILES

    @functools.partial(
        pl.kernel,
        out_type=jax.ShapeDtypeStruct((NUM_SC, n_pad, DEG_W), jnp.float32),
        mesh=_sc_mesh(),
        compiler_params=_SC_PARAMS,
        scratch_types=[
            pltpu.VMEM((nchunk, c), jnp.int32),
            pltpu.VMEM((c, DEG_W), jnp.float32),
            pltpu.VMEM_SHARED((n_pad, DEG_W), jnp.float32),
        ],
    )
    def k(dst_hbm, ones_hbm, zeros_hbm, out_hbm, dsti_v, ones_v, acc_sh):
        cid = lax.axis_index("c")
        sid = lax.axis_index("s")
        wid = sid * NUM_SC + cid
        pltpu.sync_copy(dst_hbm.at[wid], dsti_v)
        pltpu.sync_copy(ones_hbm, ones_v)
        pltpu.sync_copy(zeros_hbm, acc_sh.at[pl.ds(sid * rpt, rpt)])
        plsc.subcore_barrier()

        def body(j, carry):
            pltpu.sync_copy(ones_v, acc_sh.at[dsti_v.at[j]], add=True)
            return carry

        lax.fori_loop(0, nchunk, body, 0)
        plsc.subcore_barrier()
        pltpu.sync_copy(
            acc_sh.at[pl.ds(sid * rpt, rpt)],
            out_hbm.at[cid, pl.ds(sid * rpt, rpt)],
        )

    ones = jnp.ones((c, DEG_W), jnp.float32)
    zeros = jnp.zeros((rpt, DEG_W), jnp.float32)
    return k(dst3, ones, zeros)


def _sc_aggregate(hs, src3, dst3, n_pad):
    d = hs[0].shape[1]
    nplanes = len(hs)
    _, nchunk, c = src3.shape
    rpt = n_pad // NUM_TILES

    @functools.partial(
        pl.kernel,
        out_type=[jax.ShapeDtypeStruct((NUM_SC, n_pad, d), jnp.float32)
                  for _ in range(nplanes)],
        mesh=_sc_mesh(),
        compiler_params=_SC_PARAMS,
        scratch_types=[
            pltpu.VMEM((nchunk, c), jnp.int32),
            pltpu.VMEM((nchunk, c), jnp.int32),
            pltpu.VMEM((2, c, d), jnp.float32),
            pltpu.VMEM_SHARED((n_pad, d), jnp.float32),
            pltpu.SemaphoreType.DMA,
            pltpu.SemaphoreType.DMA,
        ],
    )
    def k(*refs):
        h_hbms = refs[:nplanes]
        src_hbm, dst_hbm, zeros_hbm = refs[nplanes:nplanes + 3]
        out_hbms = refs[nplanes + 3:2 * nplanes + 3]
        srci_v, dsti_v, rows_v, acc_sh, sem0, sem1 = refs[2 * nplanes + 3:]
        sems = (sem0, sem1)
        cid = lax.axis_index("c")
        sid = lax.axis_index("s")
        wid = sid * NUM_SC + cid
        pltpu.sync_copy(src_hbm.at[wid], srci_v)
        pltpu.sync_copy(dst_hbm.at[wid], dsti_v)

        for h_hbm, out_hbm in zip(h_hbms, out_hbms):
            pltpu.sync_copy(zeros_hbm, acc_sh.at[pl.ds(sid * rpt, rpt)])
            pltpu.async_copy(h_hbm.at[srci_v.at[0]], rows_v.at[0], sem0)
            pltpu.async_copy(h_hbm.at[srci_v.at[1]], rows_v.at[1], sem1)
            plsc.subcore_barrier()

            def body(jh, carry, h_hbm=h_hbm):
                for b in range(2):
                    jj = 2 * jh + b
                    pltpu.make_async_copy(
                        h_hbm.at[srci_v.at[jj]], rows_v.at[b], sems[b]
                    ).wait()
                    pltpu.sync_copy(rows_v.at[b], acc_sh.at[dsti_v.at[jj]],
                                    add=True)

                    @pl.when(jj + 2 < nchunk)
                    def _():
                        pltpu.async_copy(
                            h_hbm.at[srci_v.at[jj + 2]], rows_v.at[b], sems[b]
                        )
                return carry

            lax.fori_loop(0, nchunk // 2, body, 0)
            plsc.subcore_barrier()
            pltpu.sync_copy(
                acc_sh.at[pl.ds(sid * rpt, rpt)],
                out_hbm.at[cid, pl.ds(sid * rpt, rpt)],
            )

    zeros = jnp.zeros((rpt, d), jnp.float32)
    outs = k(*hs, src3, dst3, zeros)
    return list(outs) if isinstance(outs, (list, tuple)) else [outs]


def _tc_layer1(x, w1, d0, d1, blk):
    n, d_in = x.shape
    d_h = w1.shape[1]
    dh2 = d_h // 2

    def body(x_ref, w_ref, d0_ref, d1_ref, hp0_ref, hp1_ref, dis_ref):
        deg = d0_ref[...] + d1_ref[...] + 1.0
        dis = lax.rsqrt(deg)
        h = jnp.dot(x_ref[...], w_ref[...],
                    preferred_element_type=jnp.float32,
                    precision=lax.Precision.HIGHEST)
        hp = h * dis
        hp0_ref[...] = hp[:, :dh2]
        hp1_ref[...] = hp[:, dh2:]
        dis_ref[...] = dis

    return pl.pallas_call(
        body,
        grid=(n // blk,),
        in_specs=[
            pl.BlockSpec((blk, d_in), lambda i: (i, 0)),
            pl.BlockSpec((d_in, d_h), lambda i: (0, 0)),
            pl.BlockSpec((blk, 1), lambda i: (i, 0)),
            pl.BlockSpec((blk, 1), lambda i: (i, 0)),
        ],
        out_specs=[
            pl.BlockSpec((blk, dh2), lambda i: (i, 0)),
            pl.BlockSpec((blk, dh2), lambda i: (i, 0)),
            pl.BlockSpec((blk, 1), lambda i: (i, 0)),
        ],
        out_shape=[
            jax.ShapeDtypeStruct((n, dh2), jnp.float32),
            jax.ShapeDtypeStruct((n, dh2), jnp.float32),
            jax.ShapeDtypeStruct((n, 1), jnp.float32),
        ],
    )(x, w1, d0, d1)


def _tc_layer2(p0, p1, hp0, hp1, dis, b1, w2, blk):
    n, dh2 = hp0.shape
    d_o = w2.shape[1]

    def body(p0_ref, p1_ref, hp0_ref, hp1_ref, dis_ref, b1_ref, w2_ref,
             gp_ref):
        dis = dis_ref[...]
        za = dis * (p0_ref[0] + p0_ref[1] + hp0_ref[...]) + b1_ref[:, :dh2]
        zb = dis * (p1_ref[0] + p1_ref[1] + hp1_ref[...]) + b1_ref[:, dh2:]
        za = jnp.maximum(za, 0.0)
        zb = jnp.maximum(zb, 0.0)
        w2 = w2_ref[...]
        g = (jnp.dot(za, w2[:dh2],
                     preferred_element_type=jnp.float32,
                     precision=lax.Precision.HIGHEST)
             + jnp.dot(zb, w2[dh2:],
                       preferred_element_type=jnp.float32,
                       precision=lax.Precision.HIGHEST))
        gp_ref[...] = g * dis

    return pl.pallas_call(
        body,
        grid=(n // blk,),
        in_specs=[
            pl.BlockSpec((NUM_SC, blk, dh2), lambda i: (0, i, 0)),
            pl.BlockSpec((NUM_SC, blk, dh2), lambda i: (0, i, 0)),
            pl.BlockSpec((blk, dh2), lambda i: (i, 0)),
            pl.BlockSpec((blk, dh2), lambda i: (i, 0)),
            pl.BlockSpec((blk, 1), lambda i: (i, 0)),
            pl.BlockSpec((1, 2 * dh2), lambda i: (0, 0)),
            pl.BlockSpec((2 * dh2, d_o), lambda i: (0, 0)),
        ],
        out_specs=pl.BlockSpec((blk, d_o), lambda i: (i, 0)),
        out_shape=jax.ShapeDtypeStruct((n, d_o), jnp.float32),
    )(p0, p1, hp0, hp1, dis, b1, w2)


def _tc_final(q, gp, dis, b2, blk):
    n, d_o = gp.shape

    def body(q_ref, gp_ref, dis_ref, b2_ref, out_ref):
        s = q_ref[0] + q_ref[1] + gp_ref[...]
        out_ref[...] = dis_ref[...] * s + b2_ref[...]

    return pl.pallas_call(
        body,
        grid=(n // blk,),
        in_specs=[
            pl.BlockSpec((NUM_SC, blk, d_o), lambda i: (0, i, 0)),
            pl.BlockSpec((blk, d_o), lambda i: (i, 0)),
            pl.BlockSpec((blk, 1), lambda i: (i, 0)),
            pl.BlockSpec((1, d_o), lambda i: (0, 0)),
        ],
        out_specs=pl.BlockSpec((blk, d_o), lambda i: (i, 0)),
        out_shape=jax.ShapeDtypeStruct((n, d_o), jnp.float32),
    )(q, gp, dis, b2)


def kernel(x, edge_index, W1, b1, W2, b2):
    n, _ = x.shape
    e = edge_index.shape[1]

    epw = e // NUM_WORKERS
    c = 100 if epw % 100 == 0 else 80
    nchunk = epw // c
    assert epw * NUM_WORKERS == e and nchunk * c == epw and nchunk % 2 == 0

    ei = edge_index.astype(jnp.int32)
    src3 = ei[0].reshape(NUM_WORKERS, nchunk, c)
    dst3 = ei[1].reshape(NUM_WORKERS, nchunk, c)

    n_pad = ((n + NUM_TILES * 8 - 1) // (NUM_TILES * 8)) * (NUM_TILES * 8)

    degp = _sc_degree(dst3, n_pad)
    d0 = degp[0, :n, 0:1]
    d1 = degp[1, :n, 0:1]

    blk = 1000 if n % 1000 == 0 else 8
    hp0, hp1, dis = _tc_layer1(x, W1, d0, d1, blk)
    p0, p1 = _sc_aggregate([hp0, hp1], src3, dst3, n_pad)
    gp = _tc_layer2(p0, p1, hp0, hp1, dis, b1.reshape(1, -1), W2, blk)
    (q,) = _sc_aggregate([gp], src3, dst3, n_pad)
    return _tc_final(q, gp, dis, b2.reshape(1, -1), blk)


# speedup vs baseline: 28.2653x; 1.0577x over previous
import functools

import jax
import jax.numpy as jnp
from jax import lax
from jax.experimental import pallas as pl
from jax.experimental.pallas import tpu as pltpu
from jax.experimental.pallas import tpu_sc as plsc

NUM_SC = 2
NUM_TILES = 16
NUM_WORKERS = NUM_SC * NUM_TILES
DEG_W = 8


def _sc_mesh():
    return plsc.VectorSubcoreMesh(core_axis_name="c", subcore_axis_name="s")


_SC_PARAMS = pltpu.CompilerParams(use_tc_tiling_on_sc=False)


def _sc_degree(dst3, n_pad):
    _, nchunk, c = dst3.shape
    rpt = n_pad // NUM_TILES

    @functools.partial(
        pl.kernel,
        out_type=jax.ShapeDtypeStruct((NUM_SC, n_pad, DEG_W), jnp.float32),
        mesh=_sc_mesh(),
        compiler_params=_SC_PARAMS,
        scratch_types=[
            pltpu.VMEM((nchunk, c), jnp.int32),
            pltpu.VMEM((c, DEG_W), jnp.float32),
            pltpu.VMEM_SHARED((n_pad, DEG_W), jnp.float32),
        ],
    )
    def k(dst_hbm, ones_hbm, zeros_hbm, out_hbm, dsti_v, ones_v, acc_sh):
        cid = lax.axis_index("c")
        sid = lax.axis_index("s")
        wid = sid * NUM_SC + cid
        pltpu.sync_copy(dst_hbm.at[wid], dsti_v)
        pltpu.sync_copy(ones_hbm, ones_v)
        pltpu.sync_copy(zeros_hbm, acc_sh.at[pl.ds(sid * rpt, rpt)])
        plsc.subcore_barrier()

        def body(j, carry):
            pltpu.sync_copy(ones_v, acc_sh.at[dsti_v.at[j]], add=True)
            return carry

        lax.fori_loop(0, nchunk, body, 0)
        plsc.subcore_barrier()
        pltpu.sync_copy(
            acc_sh.at[pl.ds(sid * rpt, rpt)],
            out_hbm.at[cid, pl.ds(sid * rpt, rpt)],
        )

    ones = jnp.ones((c, DEG_W), jnp.float32)
    zeros = jnp.zeros((rpt, DEG_W), jnp.float32)
    return k(dst3, ones, zeros)


def _sc_aggregate(hs, src3, dst3, n_pad):
    d = hs[0].shape[1]
    nplanes = len(hs)
    _, nchunk, c = src3.shape
    rpt = n_pad // NUM_TILES

    @functools.partial(
        pl.kernel,
        out_type=[jax.ShapeDtypeStruct((NUM_SC, n_pad, d), jnp.float32)
                  for _ in range(nplanes)],
        mesh=_sc_mesh(),
        compiler_params=_SC_PARAMS,
        scratch_types=[
            pltpu.VMEM((nchunk, c), jnp.int32),
            pltpu.VMEM((nchunk, c), jnp.int32),
            pltpu.VMEM((4, c, d), jnp.float32),
            pltpu.VMEM_SHARED((n_pad, d), jnp.float32),
            [pltpu.SemaphoreType.DMA] * 4,
            [pltpu.SemaphoreType.DMA] * 4,
        ],
    )
    def k(*refs):
        h_hbms = refs[:nplanes]
        src_hbm, dst_hbm, zeros_hbm = refs[nplanes:nplanes + 3]
        out_hbms = refs[nplanes + 3:2 * nplanes + 3]
        srci_v, dsti_v, rows_v, acc_sh, gsem, ssem = refs[2 * nplanes + 3:]
        cid = lax.axis_index("c")
        sid = lax.axis_index("s")
        wid = sid * NUM_SC + cid
        pltpu.sync_copy(src_hbm.at[wid], srci_v)
        pltpu.sync_copy(dst_hbm.at[wid], dsti_v)

        def gath(h_hbm, jj, b):
            pltpu.async_copy(h_hbm.at[srci_v.at[jj]], rows_v.at[b], gsem[b])

        def gath_wait(h_hbm, jj, b):
            pltpu.make_async_copy(
                h_hbm.at[srci_v.at[jj]], rows_v.at[b], gsem[b]).wait()

        def scat(jj, b):
            pltpu.async_copy(rows_v.at[b], acc_sh.at[dsti_v.at[jj]], ssem[b],
                             add=True)

        def scat_wait(jj, b):
            pltpu.make_async_copy(
                rows_v.at[b], acc_sh.at[dsti_v.at[jj]], ssem[b]).wait()

        for h_hbm, out_hbm in zip(h_hbms, out_hbms):
            pltpu.sync_copy(zeros_hbm, acc_sh.at[pl.ds(sid * rpt, rpt)])
            gath(h_hbm, 0, 0)
            gath(h_hbm, 1, 1)
            plsc.subcore_barrier()

            for b in range(4):
                gath_wait(h_hbm, b, b % 4)
                scat(b, b % 4)
                b2 = (b + 2) % 4
                if b >= 2:
                    scat_wait(b - 2, b2)
                gath(h_hbm, b + 2, b2)

            def body(jh, carry, h_hbm=h_hbm):
                for b in range(4):
                    jj = 4 * jh + b
                    gath_wait(h_hbm, jj, b)
                    scat(jj, b)
                    b2 = (b + 2) % 4

                    @pl.when(jj + 2 < nchunk)
                    def _():
                        scat_wait(jj - 2, b2)
                        gath(h_hbm, jj + 2, b2)
                return carry

            lax.fori_loop(1, nchunk // 4, body, 0)
            for t in range(4, 0, -1):
                scat_wait(nchunk - t, (nchunk - t) % 4)
            plsc.subcore_barrier()
            pltpu.sync_copy(
                acc_sh.at[pl.ds(sid * rpt, rpt)],
                out_hbm.at[cid, pl.ds(sid * rpt, rpt)],
            )

    zeros = jnp.zeros((rpt, d), jnp.float32)
    outs = k(*hs, src3, dst3, zeros)
    return list(outs) if isinstance(outs, (list, tuple)) else [outs]


def _tc_layer1(x, w1, d0, d1, blk):
    n, d_in = x.shape
    d_h = w1.shape[1]
    dh2 = d_h // 2

    def body(x_ref, w_ref, d0_ref, d1_ref, hp0_ref, hp1_ref, dis_ref):
        deg = d0_ref[...] + d1_ref[...] + 1.0
        dis = lax.rsqrt(deg)
        h = jnp.dot(x_ref[...], w_ref[...],
                    preferred_element_type=jnp.float32,
                    precision=lax.Precision.HIGHEST)
        hp = h * dis
        hp0_ref[...] = hp[:, :dh2]
        hp1_ref[...] = hp[:, dh2:]
        dis_ref[...] = dis

    return pl.pallas_call(
        body,
        grid=(n // blk,),
        in_specs=[
            pl.BlockSpec((blk, d_in), lambda i: (i, 0)),
            pl.BlockSpec((d_in, d_h), lambda i: (0, 0)),
            pl.BlockSpec((blk, 1), lambda i: (i, 0)),
            pl.BlockSpec((blk, 1), lambda i: (i, 0)),
        ],
        out_specs=[
            pl.BlockSpec((blk, dh2), lambda i: (i, 0)),
            pl.BlockSpec((blk, dh2), lambda i: (i, 0)),
            pl.BlockSpec((blk, 1), lambda i: (i, 0)),
        ],
        out_shape=[
            jax.ShapeDtypeStruct((n, dh2), jnp.float32),
            jax.ShapeDtypeStruct((n, dh2), jnp.float32),
            jax.ShapeDtypeStruct((n, 1), jnp.float32),
        ],
    )(x, w1, d0, d1)


def _tc_layer2(p0, p1, hp0, hp1, dis, b1, w2, blk):
    n, dh2 = hp0.shape
    d_o = w2.shape[1]

    def body(p0_ref, p1_ref, hp0_ref, hp1_ref, dis_ref, b1_ref, w2_ref,
             gp_ref):
        dis = dis_ref[...]
        za = dis * (p0_ref[0] + p0_ref[1] + hp0_ref[...]) + b1_ref[:, :dh2]
        zb = dis * (p1_ref[0] + p1_ref[1] + hp1_ref[...]) + b1_ref[:, dh2:]
        za = jnp.maximum(za, 0.0)
        zb = jnp.maximum(zb, 0.0)
        w2 = w2_ref[...]
        g = (jnp.dot(za, w2[:dh2],
                     preferred_element_type=jnp.float32,
                     precision=lax.Precision.HIGHEST)
             + jnp.dot(zb, w2[dh2:],
                       preferred_element_type=jnp.float32,
                       precision=lax.Precision.HIGHEST))
        gp_ref[...] = g * dis

    return pl.pallas_call(
        body,
        grid=(n // blk,),
        in_specs=[
            pl.BlockSpec((NUM_SC, blk, dh2), lambda i: (0, i, 0)),
            pl.BlockSpec((NUM_SC, blk, dh2), lambda i: (0, i, 0)),
            pl.BlockSpec((blk, dh2), lambda i: (i, 0)),
            pl.BlockSpec((blk, dh2), lambda i: (i, 0)),
            pl.BlockSpec((blk, 1), lambda i: (i, 0)),
            pl.BlockSpec((1, 2 * dh2), lambda i: (0, 0)),
            pl.BlockSpec((2 * dh2, d_o), lambda i: (0, 0)),
        ],
        out_specs=pl.BlockSpec((blk, d_o), lambda i: (i, 0)),
        out_shape=jax.ShapeDtypeStruct((n, d_o), jnp.float32),
    )(p0, p1, hp0, hp1, dis, b1, w2)


def _tc_final(q, gp, dis, b2, blk):
    n, d_o = gp.shape

    def body(q_ref, gp_ref, dis_ref, b2_ref, out_ref):
        s = q_ref[0] + q_ref[1] + gp_ref[...]
        out_ref[...] = dis_ref[...] * s + b2_ref[...]

    return pl.pallas_call(
        body,
        grid=(n // blk,),
        in_specs=[
            pl.BlockSpec((NUM_SC, blk, d_o), lambda i: (0, i, 0)),
            pl.BlockSpec((blk, d_o), lambda i: (i, 0)),
            pl.BlockSpec((blk, 1), lambda i: (i, 0)),
            pl.BlockSpec((1, d_o), lambda i: (0, 0)),
        ],
        out_specs=pl.BlockSpec((blk, d_o), lambda i: (i, 0)),
        out_shape=jax.ShapeDtypeStruct((n, d_o), jnp.float32),
    )(q, gp, dis, b2)


def kernel(x, edge_index, W1, b1, W2, b2):
    n, _ = x.shape
    e = edge_index.shape[1]

    epw = e // NUM_WORKERS
    c = 100 if epw % 100 == 0 else 80
    nchunk = epw // c
    assert epw * NUM_WORKERS == e and nchunk * c == epw and nchunk % 2 == 0

    ei = edge_index.astype(jnp.int32)
    src3 = ei[0].reshape(NUM_WORKERS, nchunk, c)
    dst3 = ei[1].reshape(NUM_WORKERS, nchunk, c)

    n_pad = ((n + NUM_TILES * 8 - 1) // (NUM_TILES * 8)) * (NUM_TILES * 8)

    degp = _sc_degree(dst3, n_pad)
    d0 = degp[0, :n, 0:1]
    d1 = degp[1, :n, 0:1]

    blk = 1000 if n % 1000 == 0 else 8
    hp0, hp1, dis = _tc_layer1(x, W1, d0, d1, blk)
    p0, p1 = _sc_aggregate([hp0, hp1], src3, dst3, n_pad)
    gp = _tc_layer2(p0, p1, hp0, hp1, dis, b1.reshape(1, -1), W2, blk)
    (q,) = _sc_aggregate([gp], src3, dst3, n_pad)
    return _tc_final(q, gp, dis, b2.reshape(1, -1), blk)


# speedup vs baseline: 29.2795x; 1.0359x over previous
import functools

import jax
import jax.numpy as jnp
from jax import lax
from jax.experimental import pallas as pl
from jax.experimental.pallas import tpu as pltpu
from jax.experimental.pallas import tpu_sc as plsc

NUM_SC = 2
NUM_TILES = 16
NUM_WORKERS = NUM_SC * NUM_TILES
DEG_W = 8


def _sc_mesh():
    return plsc.VectorSubcoreMesh(core_axis_name="c", subcore_axis_name="s")


_SC_PARAMS = pltpu.CompilerParams(use_tc_tiling_on_sc=False)


def _sc_degree(dst3, n_pad):
    _, nchunk, c = dst3.shape
    rpt = n_pad // NUM_TILES

    @functools.partial(
        pl.kernel,
        out_type=jax.ShapeDtypeStruct((NUM_SC, n_pad, DEG_W), jnp.float32),
        mesh=_sc_mesh(),
        compiler_params=_SC_PARAMS,
        scratch_types=[
            pltpu.VMEM((nchunk, c), jnp.int32),
            pltpu.VMEM((c, DEG_W), jnp.float32),
            pltpu.VMEM_SHARED((n_pad, DEG_W), jnp.float32),
        ],
    )
    def k(dst_hbm, ones_hbm, zeros_hbm, out_hbm, dsti_v, ones_v, acc_sh):
        cid = lax.axis_index("c")
        sid = lax.axis_index("s")
        wid = sid * NUM_SC + cid
        pltpu.sync_copy(dst_hbm.at[wid], dsti_v)
        pltpu.sync_copy(ones_hbm, ones_v)
        pltpu.sync_copy(zeros_hbm, acc_sh.at[pl.ds(sid * rpt, rpt)])
        plsc.subcore_barrier()

        def body(j, carry):
            pltpu.sync_copy(ones_v, acc_sh.at[dsti_v.at[j]], add=True)
            return carry

        lax.fori_loop(0, nchunk, body, 0)
        plsc.subcore_barrier()
        pltpu.sync_copy(
            acc_sh.at[pl.ds(sid * rpt, rpt)],
            out_hbm.at[cid, pl.ds(sid * rpt, rpt)],
        )

    ones = jnp.ones((c, DEG_W), jnp.float32)
    zeros = jnp.zeros((rpt, DEG_W), jnp.float32)
    return k(dst3, ones, zeros)


def _sc_aggregate(hs, src3, dst3, n_pad):
    d = hs[0].shape[1]
    nplanes = len(hs)
    _, nchunk, c = src3.shape
    rpt = n_pad // NUM_TILES

    @functools.partial(
        pl.kernel,
        out_type=[jax.ShapeDtypeStruct((NUM_SC, n_pad, d), jnp.float32)
                  for _ in range(nplanes)],
        mesh=_sc_mesh(),
        compiler_params=_SC_PARAMS,
        scratch_types=[
            pltpu.VMEM((nchunk, c), jnp.int32),
            pltpu.VMEM((nchunk, c), jnp.int32),
            pltpu.VMEM((4, c, d), jnp.float32),
            pltpu.VMEM_SHARED((n_pad, d), jnp.float32),
            [pltpu.SemaphoreType.DMA] * 4,
            [pltpu.SemaphoreType.DMA] * 4,
        ],
    )
    def k(*refs):
        h_hbms = refs[:nplanes]
        src_hbm, dst_hbm, zeros_hbm = refs[nplanes:nplanes + 3]
        out_hbms = refs[nplanes + 3:2 * nplanes + 3]
        srci_v, dsti_v, rows_v, acc_sh, gsem, ssem = refs[2 * nplanes + 3:]
        cid = lax.axis_index("c")
        sid = lax.axis_index("s")
        wid = sid * NUM_SC + cid
        pltpu.sync_copy(src_hbm.at[wid], srci_v)
        pltpu.sync_copy(dst_hbm.at[wid], dsti_v)

        def gath(h_hbm, jj, b):
            pltpu.async_copy(h_hbm.at[srci_v.at[jj]], rows_v.at[b], gsem[b])

        def gath_wait(h_hbm, jj, b):
            pltpu.make_async_copy(
                h_hbm.at[srci_v.at[jj]], rows_v.at[b], gsem[b]).wait()

        def scat(jj, b):
            pltpu.async_copy(rows_v.at[b], acc_sh.at[dsti_v.at[jj]], ssem[b],
                             add=True)

        def scat_wait(jj, b):
            pltpu.make_async_copy(
                rows_v.at[b], acc_sh.at[dsti_v.at[jj]], ssem[b]).wait()

        for h_hbm, out_hbm in zip(h_hbms, out_hbms):
            pltpu.sync_copy(zeros_hbm, acc_sh.at[pl.ds(sid * rpt, rpt)])
            gath(h_hbm, 0, 0)
            gath(h_hbm, 1, 1)
            plsc.subcore_barrier()

            for b in range(4):
                gath_wait(h_hbm, b, b % 4)
                if b >= 1:
                    scat_wait(b - 1, (b - 1) % 4)
                scat(b, b % 4)
                gath(h_hbm, b + 2, (b + 2) % 4)

            def body(jh, carry, h_hbm=h_hbm):
                for b in range(4):
                    jj = 4 * jh + b
                    gath_wait(h_hbm, jj, b)
                    scat_wait(jj - 1, (b + 3) % 4)
                    scat(jj, b)

                    @pl.when(jj + 2 < nchunk)
                    def _():
                        gath(h_hbm, jj + 2, (b + 2) % 4)
                return carry

            lax.fori_loop(1, nchunk // 4, body, 0)
            scat_wait(nchunk - 1, (nchunk - 1) % 4)
            plsc.subcore_barrier()
            pltpu.sync_copy(
                acc_sh.at[pl.ds(sid * rpt, rpt)],
                out_hbm.at[cid, pl.ds(sid * rpt, rpt)],
            )

    zeros = jnp.zeros((rpt, d), jnp.float32)
    outs = k(*hs, src3, dst3, zeros)
    return list(outs) if isinstance(outs, (list, tuple)) else [outs]


def _tc_layer1(x, w1, d0, d1, blk):
    n, d_in = x.shape
    d_h = w1.shape[1]
    dh2 = d_h // 2

    def body(x_ref, w_ref, d0_ref, d1_ref, hp0_ref, hp1_ref, dis_ref):
        deg = d0_ref[...] + d1_ref[...] + 1.0
        dis = lax.rsqrt(deg)
        h = jnp.dot(x_ref[...], w_ref[...],
                    preferred_element_type=jnp.float32,
                    precision=lax.Precision.HIGHEST)
        hp = h * dis
        hp0_ref[...] = hp[:, :dh2]
        hp1_ref[...] = hp[:, dh2:]
        dis_ref[...] = dis

    return pl.pallas_call(
        body,
        grid=(n // blk,),
        in_specs=[
            pl.BlockSpec((blk, d_in), lambda i: (i, 0)),
            pl.BlockSpec((d_in, d_h), lambda i: (0, 0)),
            pl.BlockSpec((blk, 1), lambda i: (i, 0)),
            pl.BlockSpec((blk, 1), lambda i: (i, 0)),
        ],
        out_specs=[
            pl.BlockSpec((blk, dh2), lambda i: (i, 0)),
            pl.BlockSpec((blk, dh2), lambda i: (i, 0)),
            pl.BlockSpec((blk, 1), lambda i: (i, 0)),
        ],
        out_shape=[
            jax.ShapeDtypeStruct((n, dh2), jnp.float32),
            jax.ShapeDtypeStruct((n, dh2), jnp.float32),
            jax.ShapeDtypeStruct((n, 1), jnp.float32),
        ],
    )(x, w1, d0, d1)


def _tc_layer2(p0, p1, hp0, hp1, dis, b1, w2, blk):
    n, dh2 = hp0.shape
    d_o = w2.shape[1]

    def body(p0_ref, p1_ref, hp0_ref, hp1_ref, dis_ref, b1_ref, w2_ref,
             gp_ref):
        dis = dis_ref[...]
        za = dis * (p0_ref[0] + p0_ref[1] + hp0_ref[...]) + b1_ref[:, :dh2]
        zb = dis * (p1_ref[0] + p1_ref[1] + hp1_ref[...]) + b1_ref[:, dh2:]
        za = jnp.maximum(za, 0.0)
        zb = jnp.maximum(zb, 0.0)
        w2 = w2_ref[...]
        g = (jnp.dot(za, w2[:dh2],
                     preferred_element_type=jnp.float32,
                     precision=lax.Precision.HIGHEST)
             + jnp.dot(zb, w2[dh2:],
                       preferred_element_type=jnp.float32,
                       precision=lax.Precision.HIGHEST))
        gp_ref[...] = g * dis

    return pl.pallas_call(
        body,
        grid=(n // blk,),
        in_specs=[
            pl.BlockSpec((NUM_SC, blk, dh2), lambda i: (0, i, 0)),
            pl.BlockSpec((NUM_SC, blk, dh2), lambda i: (0, i, 0)),
            pl.BlockSpec((blk, dh2), lambda i: (i, 0)),
            pl.BlockSpec((blk, dh2), lambda i: (i, 0)),
            pl.BlockSpec((blk, 1), lambda i: (i, 0)),
            pl.BlockSpec((1, 2 * dh2), lambda i: (0, 0)),
            pl.BlockSpec((2 * dh2, d_o), lambda i: (0, 0)),
        ],
        out_specs=pl.BlockSpec((blk, d_o), lambda i: (i, 0)),
        out_shape=jax.ShapeDtypeStruct((n, d_o), jnp.float32),
    )(p0, p1, hp0, hp1, dis, b1, w2)


def _tc_final(q, gp, dis, b2, blk):
    n, d_o = gp.shape

    def body(q_ref, gp_ref, dis_ref, b2_ref, out_ref):
        s = q_ref[0] + q_ref[1] + gp_ref[...]
        out_ref[...] = dis_ref[...] * s + b2_ref[...]

    return pl.pallas_call(
        body,
        grid=(n // blk,),
        in_specs=[
            pl.BlockSpec((NUM_SC, blk, d_o), lambda i: (0, i, 0)),
            pl.BlockSpec((blk, d_o), lambda i: (i, 0)),
            pl.BlockSpec((blk, 1), lambda i: (i, 0)),
            pl.BlockSpec((1, d_o), lambda i: (0, 0)),
        ],
        out_specs=pl.BlockSpec((blk, d_o), lambda i: (i, 0)),
        out_shape=jax.ShapeDtypeStruct((n, d_o), jnp.float32),
    )(q, gp, dis, b2)


def kernel(x, edge_index, W1, b1, W2, b2):
    n, _ = x.shape
    e = edge_index.shape[1]

    epw = e // NUM_WORKERS
    c = 100 if epw % 100 == 0 else 80
    nchunk = epw // c
    assert epw * NUM_WORKERS == e and nchunk * c == epw and nchunk % 2 == 0

    ei = edge_index.astype(jnp.int32)
    src3 = ei[0].reshape(NUM_WORKERS, nchunk, c)
    dst3 = ei[1].reshape(NUM_WORKERS, nchunk, c)

    n_pad = ((n + NUM_TILES * 8 - 1) // (NUM_TILES * 8)) * (NUM_TILES * 8)

    degp = _sc_degree(dst3, n_pad)
    d0 = degp[0, :n, 0:1]
    d1 = degp[1, :n, 0:1]

    blk = 1000 if n % 1000 == 0 else 8
    hp0, hp1, dis = _tc_layer1(x, W1, d0, d1, blk)
    p0, p1 = _sc_aggregate([hp0, hp1], src3, dst3, n_pad)
    gp = _tc_layer2(p0, p1, hp0, hp1, dis, b1.reshape(1, -1), W2, blk)
    (q,) = _sc_aggregate([gp], src3, dst3, n_pad)
    return _tc_final(q, gp, dis, b2.reshape(1, -1), blk)


# speedup vs baseline: 32.3070x; 1.1034x over previous
import functools

import jax
import jax.numpy as jnp
from jax import lax
from jax.experimental import pallas as pl
from jax.experimental.pallas import tpu as pltpu
from jax.experimental.pallas import tpu_sc as plsc

NUM_SC = 2
NUM_TILES = 16
NUM_WORKERS = NUM_SC * NUM_TILES
DEG_W = 8


def _sc_mesh():
    return plsc.VectorSubcoreMesh(core_axis_name="c", subcore_axis_name="s")


_SC_PARAMS = pltpu.CompilerParams(use_tc_tiling_on_sc=False)


def _sc_degree(ei3, n_pad):
    _, _, nchunk, c = ei3.shape
    rpt = n_pad // NUM_TILES

    @functools.partial(
        pl.kernel,
        out_type=jax.ShapeDtypeStruct((NUM_SC, n_pad, DEG_W), jnp.float32),
        mesh=_sc_mesh(),
        compiler_params=_SC_PARAMS,
        scratch_types=[
            pltpu.VMEM((nchunk, c), jnp.int32),
            pltpu.VMEM((c, DEG_W), jnp.float32),
            pltpu.VMEM_SHARED((n_pad, DEG_W), jnp.float32),
        ],
    )
    def k(ei_hbm, ones_hbm, zeros_hbm, out_hbm, dsti_v, ones_v, acc_sh):
        cid = lax.axis_index("c")
        sid = lax.axis_index("s")
        wid = sid * NUM_SC + cid
        pltpu.sync_copy(ei_hbm.at[1, wid], dsti_v)
        pltpu.sync_copy(ones_hbm, ones_v)
        pltpu.sync_copy(zeros_hbm, acc_sh.at[pl.ds(sid * rpt, rpt)])
        plsc.subcore_barrier()

        def body(j, carry):
            pltpu.sync_copy(ones_v, acc_sh.at[dsti_v.at[j]], add=True)
            return carry

        lax.fori_loop(0, nchunk, body, 0)
        plsc.subcore_barrier()
        pltpu.sync_copy(
            acc_sh.at[pl.ds(sid * rpt, rpt)],
            out_hbm.at[cid, pl.ds(sid * rpt, rpt)],
        )

    ones = jnp.ones((c, DEG_W), jnp.float32)
    zeros = jnp.zeros((rpt, DEG_W), jnp.float32)
    return k(ei3, ones, zeros)


def _sc_aggregate(hs, ei3, n_pad):
    d = hs[0].shape[1]
    nplanes = len(hs)
    _, _, nchunk, c = ei3.shape
    rpt = n_pad // NUM_TILES

    @functools.partial(
        pl.kernel,
        out_type=[jax.ShapeDtypeStruct((NUM_SC, n_pad, d), jnp.float32)
                  for _ in range(nplanes)],
        mesh=_sc_mesh(),
        compiler_params=_SC_PARAMS,
        scratch_types=[
            pltpu.VMEM((nchunk, c), jnp.int32),
            pltpu.VMEM((nchunk, c), jnp.int32),
            pltpu.VMEM((4, c, d), jnp.float32),
            pltpu.VMEM_SHARED((n_pad, d), jnp.float32),
            [pltpu.SemaphoreType.DMA] * 4,
            [pltpu.SemaphoreType.DMA] * 4,
        ],
    )
    def k(*refs):
        h_hbms = refs[:nplanes]
        ei_hbm, zeros_hbm = refs[nplanes:nplanes + 2]
        out_hbms = refs[nplanes + 2:2 * nplanes + 2]
        srci_v, dsti_v, rows_v, acc_sh, gsem, ssem = refs[2 * nplanes + 2:]
        cid = lax.axis_index("c")
        sid = lax.axis_index("s")
        wid = sid * NUM_SC + cid
        pltpu.sync_copy(ei_hbm.at[0, wid], srci_v)
        pltpu.sync_copy(ei_hbm.at[1, wid], dsti_v)

        def gath(h_hbm, jj, b):
            pltpu.async_copy(h_hbm.at[srci_v.at[jj]], rows_v.at[b], gsem[b])

        def gath_wait(h_hbm, jj, b):
            pltpu.make_async_copy(
                h_hbm.at[srci_v.at[jj]], rows_v.at[b], gsem[b]).wait()

        def scat(jj, b):
            pltpu.async_copy(rows_v.at[b], acc_sh.at[dsti_v.at[jj]], ssem[b],
                             add=True)

        def scat_wait(jj, b):
            pltpu.make_async_copy(
                rows_v.at[b], acc_sh.at[dsti_v.at[jj]], ssem[b]).wait()

        for h_hbm, out_hbm in zip(h_hbms, out_hbms):
            pltpu.sync_copy(zeros_hbm, acc_sh.at[pl.ds(sid * rpt, rpt)])
            gath(h_hbm, 0, 0)
            gath(h_hbm, 1, 1)
            plsc.subcore_barrier()

            for b in range(4):
                gath_wait(h_hbm, b, b % 4)
                if b >= 1:
                    scat_wait(b - 1, (b - 1) % 4)
                scat(b, b % 4)
                gath(h_hbm, b + 2, (b + 2) % 4)

            def body(jh, carry, h_hbm=h_hbm):
                for b in range(4):
                    jj = 4 * jh + b
                    gath_wait(h_hbm, jj, b)
                    scat_wait(jj - 1, (b + 3) % 4)
                    scat(jj, b)

                    @pl.when(jj + 2 < nchunk)
                    def _():
                        gath(h_hbm, jj + 2, (b + 2) % 4)
                return carry

            lax.fori_loop(1, nchunk // 4, body, 0)
            scat_wait(nchunk - 1, (nchunk - 1) % 4)
            plsc.subcore_barrier()
            pltpu.sync_copy(
                acc_sh.at[pl.ds(sid * rpt, rpt)],
                out_hbm.at[cid, pl.ds(sid * rpt, rpt)],
            )

    zeros = jnp.zeros((rpt, d), jnp.float32)
    outs = k(*hs, ei3, zeros)
    return list(outs) if isinstance(outs, (list, tuple)) else [outs]


def _tc_layer1(x, w1, degp, blk):
    n, d_in = x.shape
    d_h = w1.shape[1]
    dh2 = d_h // 2

    def body(x_ref, w_ref, degp_ref, hp0_ref, hp1_ref, dis_ref):
        deg = degp_ref[0, :, 0:1] + degp_ref[1, :, 0:1] + 1.0
        dis = lax.rsqrt(deg)
        h = jnp.dot(x_ref[...], w_ref[...],
                    preferred_element_type=jnp.float32,
                    precision=lax.Precision.HIGHEST)
        hp = h * dis
        hp0_ref[...] = hp[:, :dh2]
        hp1_ref[...] = hp[:, dh2:]
        dis_ref[...] = dis

    return pl.pallas_call(
        body,
        grid=(n // blk,),
        in_specs=[
            pl.BlockSpec((blk, d_in), lambda i: (i, 0)),
            pl.BlockSpec((d_in, d_h), lambda i: (0, 0)),
            pl.BlockSpec((NUM_SC, blk, DEG_W), lambda i: (0, i, 0)),
        ],
        out_specs=[
            pl.BlockSpec((blk, dh2), lambda i: (i, 0)),
            pl.BlockSpec((blk, dh2), lambda i: (i, 0)),
            pl.BlockSpec((blk, 1), lambda i: (i, 0)),
        ],
        out_shape=[
            jax.ShapeDtypeStruct((n, dh2), jnp.float32),
            jax.ShapeDtypeStruct((n, dh2), jnp.float32),
            jax.ShapeDtypeStruct((n, 1), jnp.float32),
        ],
    )(x, w1, degp)


def _tc_layer2(p0, p1, hp0, hp1, dis, b1, w2, blk):
    n, dh2 = hp0.shape
    d_o = w2.shape[1]

    def body(p0_ref, p1_ref, hp0_ref, hp1_ref, dis_ref, b1_ref, w2_ref,
             gp_ref):
        dis = dis_ref[...]
        za = dis * (p0_ref[0] + p0_ref[1] + hp0_ref[...]) + b1_ref[:, :dh2]
        zb = dis * (p1_ref[0] + p1_ref[1] + hp1_ref[...]) + b1_ref[:, dh2:]
        za = jnp.maximum(za, 0.0)
        zb = jnp.maximum(zb, 0.0)
        w2 = w2_ref[...]
        g = (jnp.dot(za, w2[:dh2],
                     preferred_element_type=jnp.float32,
                     precision=lax.Precision.HIGHEST)
             + jnp.dot(zb, w2[dh2:],
                       preferred_element_type=jnp.float32,
                       precision=lax.Precision.HIGHEST))
        gp_ref[...] = g * dis

    return pl.pallas_call(
        body,
        grid=(n // blk,),
        in_specs=[
            pl.BlockSpec((NUM_SC, blk, dh2), lambda i: (0, i, 0)),
            pl.BlockSpec((NUM_SC, blk, dh2), lambda i: (0, i, 0)),
            pl.BlockSpec((blk, dh2), lambda i: (i, 0)),
            pl.BlockSpec((blk, dh2), lambda i: (i, 0)),
            pl.BlockSpec((blk, 1), lambda i: (i, 0)),
            pl.BlockSpec((1, 2 * dh2), lambda i: (0, 0)),
            pl.BlockSpec((2 * dh2, d_o), lambda i: (0, 0)),
        ],
        out_specs=pl.BlockSpec((blk, d_o), lambda i: (i, 0)),
        out_shape=jax.ShapeDtypeStruct((n, d_o), jnp.float32),
    )(p0, p1, hp0, hp1, dis, b1, w2)


def _tc_final(q, gp, dis, b2, blk):
    n, d_o = gp.shape

    def body(q_ref, gp_ref, dis_ref, b2_ref, out_ref):
        s = q_ref[0] + q_ref[1] + gp_ref[...]
        out_ref[...] = dis_ref[...] * s + b2_ref[...]

    return pl.pallas_call(
        body,
        grid=(n // blk,),
        in_specs=[
            pl.BlockSpec((NUM_SC, blk, d_o), lambda i: (0, i, 0)),
            pl.BlockSpec((blk, d_o), lambda i: (i, 0)),
            pl.BlockSpec((blk, 1), lambda i: (i, 0)),
            pl.BlockSpec((1, d_o), lambda i: (0, 0)),
        ],
        out_specs=pl.BlockSpec((blk, d_o), lambda i: (i, 0)),
        out_shape=jax.ShapeDtypeStruct((n, d_o), jnp.float32),
    )(q, gp, dis, b2)


def kernel(x, edge_index, W1, b1, W2, b2):
    n, _ = x.shape
    e = edge_index.shape[1]

    epw = e // NUM_WORKERS
    c = 125 if epw % 125 == 0 else 100
    nchunk = epw // c
    assert epw * NUM_WORKERS == e and nchunk * c == epw and nchunk % 4 == 0

    ei3 = edge_index.astype(jnp.int32).reshape(2, NUM_WORKERS, nchunk, c)

    n_pad = ((n + NUM_TILES * 8 - 1) // (NUM_TILES * 8)) * (NUM_TILES * 8)

    degp = _sc_degree(ei3, n_pad)

    blk = 1000 if n % 1000 == 0 else 8
    hp0, hp1, dis = _tc_layer1(x, W1, degp, blk)
    p0, p1 = _sc_aggregate([hp0, hp1], ei3, n_pad)
    gp = _tc_layer2(p0, p1, hp0, hp1, dis, b1.reshape(1, -1), W2, blk)
    (q,) = _sc_aggregate([gp], ei3, n_pad)
    return _tc_final(q, gp, dis, b2.reshape(1, -1), blk)


# speedup vs baseline: 34.3392x; 1.0629x over previous
import functools

import jax
import jax.numpy as jnp
from jax import lax
from jax.experimental import pallas as pl
from jax.experimental.pallas import tpu as pltpu
from jax.experimental.pallas import tpu_sc as plsc

NUM_SC = 2
NUM_TILES = 16
NUM_WORKERS = NUM_SC * NUM_TILES
DEG_W = 8


def _sc_mesh():
    return plsc.VectorSubcoreMesh(core_axis_name="c", subcore_axis_name="s")


_SC_PARAMS = pltpu.CompilerParams(use_tc_tiling_on_sc=False)


def _sc_degree(ei4, n_pad):
    _, _, nchunk, c = ei4.shape
    rpt = n_pad // NUM_TILES

    @functools.partial(
        pl.kernel,
        out_type=jax.ShapeDtypeStruct((NUM_SC, n_pad, DEG_W), jnp.float32),
        mesh=_sc_mesh(),
        compiler_params=_SC_PARAMS,
        scratch_types=[
            pltpu.VMEM((nchunk, c), jnp.int32),
            pltpu.VMEM((c, DEG_W), jnp.float32),
            pltpu.VMEM_SHARED((n_pad, DEG_W), jnp.float32),
        ],
    )
    def k(ei_hbm, ones_hbm, zeros_hbm, out_hbm, dsti_v, ones_v, acc_sh):
        cid = lax.axis_index("c")
        sid = lax.axis_index("s")
        wid = sid * NUM_SC + cid
        pltpu.sync_copy(ei_hbm.at[1, wid], dsti_v)
        pltpu.sync_copy(ones_hbm, ones_v)
        pltpu.sync_copy(zeros_hbm, acc_sh.at[pl.ds(sid * rpt, rpt)])
        plsc.subcore_barrier()

        def body(j, carry):
            pltpu.sync_copy(ones_v, acc_sh.at[dsti_v.at[j]], add=True)
            return carry

        lax.fori_loop(0, nchunk, body, 0)
        plsc.subcore_barrier()
        pltpu.sync_copy(
            acc_sh.at[pl.ds(sid * rpt, rpt)],
            out_hbm.at[cid, pl.ds(sid * rpt, rpt)],
        )

    ones = jnp.ones((c, DEG_W), jnp.float32)
    zeros = jnp.zeros((rpt, DEG_W), jnp.float32)
    return k(ei4, ones, zeros)


def _sc_aggregate(h2, ei4, n_pad, planes):
    d = h2.shape[1]
    nplanes = len(planes)
    out_w = d * nplanes
    _, _, nchunk, c = ei4.shape
    rpt = n_pad // NUM_TILES

    @functools.partial(
        pl.kernel,
        out_type=jax.ShapeDtypeStruct((NUM_SC, n_pad, out_w), jnp.float32),
        mesh=_sc_mesh(),
        compiler_params=_SC_PARAMS,
        scratch_types=[
            pltpu.VMEM((nchunk, c), jnp.int32),
            pltpu.VMEM((nchunk, c), jnp.int32),
            pltpu.VMEM((4, c, d), jnp.float32),
            pltpu.VMEM_SHARED((n_pad, d), jnp.float32),
            [pltpu.SemaphoreType.DMA] * 4,
            [pltpu.SemaphoreType.DMA] * 4,
        ],
    )
    def k(h_hbm, ei_hbm, zeros_hbm, out_hbm,
          srci_v, dsti_v, rows_v, acc_sh, gsem, ssem):
        cid = lax.axis_index("c")
        sid = lax.axis_index("s")
        wid = sid * NUM_SC + cid
        pltpu.sync_copy(ei_hbm.at[1, wid], dsti_v)

        def gath(jj, b):
            pltpu.async_copy(h_hbm.at[srci_v.at[jj]], rows_v.at[b], gsem[b])

        def gath_wait(jj, b):
            pltpu.make_async_copy(
                h_hbm.at[srci_v.at[jj]], rows_v.at[b], gsem[b]).wait()

        def scat(jj, b):
            pltpu.async_copy(rows_v.at[b], acc_sh.at[dsti_v.at[jj]], ssem[b],
                             add=True)

        def scat_wait(jj, b):
            pltpu.make_async_copy(
                rows_v.at[b], acc_sh.at[dsti_v.at[jj]], ssem[b]).wait()

        for idx_plane, out_col in planes:
            pltpu.sync_copy(ei_hbm.at[idx_plane, wid], srci_v)
            pltpu.sync_copy(zeros_hbm, acc_sh.at[pl.ds(sid * rpt, rpt)])
            gath(0, 0)
            gath(1, 1)
            plsc.subcore_barrier()

            for b in range(4):
                gath_wait(b, b % 4)
                if b >= 1:
                    scat_wait(b - 1, (b - 1) % 4)
                scat(b, b % 4)
                gath(b + 2, (b + 2) % 4)

            def body(jh, carry):
                for b in range(4):
                    jj = 4 * jh + b
                    gath_wait(jj, b)
                    scat_wait(jj - 1, (b + 3) % 4)
                    scat(jj, b)

                    @pl.when(jj + 2 < nchunk)
                    def _():
                        gath(jj + 2, (b + 2) % 4)
                return carry

            lax.fori_loop(1, nchunk // 4, body, 0)
            scat_wait(nchunk - 1, (nchunk - 1) % 4)
            plsc.subcore_barrier()
            pltpu.sync_copy(
                acc_sh.at[pl.ds(sid * rpt, rpt)],
                out_hbm.at[cid, pl.ds(sid * rpt, rpt),
                           pl.ds(out_col, d)],
            )

    zeros = jnp.zeros((rpt, d), jnp.float32)
    return k(h2, ei4, zeros)


def _tc_layer1(x, w1, degp, blk):
    n, d_in = x.shape
    d_h = w1.shape[1]

    def body(x_ref, w_ref, degp_ref, hp_ref, dis_ref):
        deg = degp_ref[0, :, 0:1] + degp_ref[1, :, 0:1] + 1.0
        dis = lax.rsqrt(deg)
        h = jnp.dot(x_ref[...], w_ref[...],
                    preferred_element_type=jnp.float32,
                    precision=lax.Precision.HIGHEST)
        hp_ref[...] = h * dis
        dis_ref[...] = dis

    return pl.pallas_call(
        body,
        grid=(n // blk,),
        in_specs=[
            pl.BlockSpec((blk, d_in), lambda i: (i, 0)),
            pl.BlockSpec((d_in, d_h), lambda i: (0, 0)),
            pl.BlockSpec((NUM_SC, blk, DEG_W), lambda i: (0, i, 0)),
        ],
        out_specs=[
            pl.BlockSpec((blk, d_h), lambda i: (i, 0)),
            pl.BlockSpec((blk, 1), lambda i: (i, 0)),
        ],
        out_shape=[
            jax.ShapeDtypeStruct((n, d_h), jnp.float32),
            jax.ShapeDtypeStruct((n, 1), jnp.float32),
        ],
    )(x, w1, degp)


def _tc_layer2(p, hp, dis, b1, w2, blk):
    n, d_h = hp.shape
    d_o = w2.shape[1]

    def body(p_ref, hp_ref, dis_ref, b1_ref, w2_ref, gp_ref):
        dis = dis_ref[...]
        z = jnp.maximum(dis * (p_ref[0] + p_ref[1] + hp_ref[...])
                        + b1_ref[...], 0.0)
        g = jnp.dot(z, w2_ref[...],
                    preferred_element_type=jnp.float32,
                    precision=lax.Precision.HIGHEST)
        gp_ref[...] = g * dis

    return pl.pallas_call(
        body,
        grid=(n // blk,),
        in_specs=[
            pl.BlockSpec((NUM_SC, blk, d_h), lambda i: (0, i, 0)),
            pl.BlockSpec((blk, d_h), lambda i: (i, 0)),
            pl.BlockSpec((blk, 1), lambda i: (i, 0)),
            pl.BlockSpec((1, d_h), lambda i: (0, 0)),
            pl.BlockSpec((d_h, d_o), lambda i: (0, 0)),
        ],
        out_specs=pl.BlockSpec((blk, d_o), lambda i: (i, 0)),
        out_shape=jax.ShapeDtypeStruct((n, d_o), jnp.float32),
    )(p, hp, dis, b1, w2)


def _tc_final(q, gp, dis, b2, blk):
    n, d_o = gp.shape

    def body(q_ref, gp_ref, dis_ref, b2_ref, out_ref):
        s = q_ref[0] + q_ref[1] + gp_ref[...]
        out_ref[...] = dis_ref[...] * s + b2_ref[...]

    return pl.pallas_call(
        body,
        grid=(n // blk,),
        in_specs=[
            pl.BlockSpec((NUM_SC, blk, d_o), lambda i: (0, i, 0)),
            pl.BlockSpec((blk, d_o), lambda i: (i, 0)),
            pl.BlockSpec((blk, 1), lambda i: (i, 0)),
            pl.BlockSpec((1, d_o), lambda i: (0, 0)),
        ],
        out_specs=pl.BlockSpec((blk, d_o), lambda i: (i, 0)),
        out_shape=jax.ShapeDtypeStruct((n, d_o), jnp.float32),
    )(q, gp, dis, b2)


def kernel(x, edge_index, W1, b1, W2, b2):
    n, _ = x.shape
    e = edge_index.shape[1]

    epw = e // NUM_WORKERS
    c = 125 if epw % 125 == 0 else 100
    nchunk = epw // c
    assert epw * NUM_WORKERS == e and nchunk * c == epw and nchunk % 4 == 0

    ei = edge_index.astype(jnp.int32)
    src = ei[0].reshape(NUM_WORKERS, nchunk, c)
    dst = ei[1].reshape(NUM_WORKERS, nchunk, c)
    ei4 = jnp.stack([src, dst, 2 * src, 2 * src + 1])

    n_pad = ((n + NUM_TILES * 8 - 1) // (NUM_TILES * 8)) * (NUM_TILES * 8)

    degp = _sc_degree(ei4, n_pad)

    blk = 1000 if n % 1000 == 0 else 8
    hp, dis = _tc_layer1(x, W1, degp, blk)
    hp2 = hp.reshape(2 * n, hp.shape[1] // 2)
    p = _sc_aggregate(hp2, ei4, n_pad, [(2, 0), (3, hp.shape[1] // 2)])
    gp = _tc_layer2(p, hp, dis, b1.reshape(1, -1), W2, blk)
    q = _sc_aggregate(gp, ei4, n_pad, [(0, 0)])
    return _tc_final(q, gp, dis, b2.reshape(1, -1), blk)
